# Initial kernel scaffold; baseline (speedup 1.0000x reference)
import jax
import jax.numpy as jnp
from jax.experimental import pallas as pl


def kernel(ContTensor, CatTensor, LabelTensor, DoseTensor, TimeDiffTensor, VTensor, VancoElTensor, PtList, LengList, embed_weight):
    raise NotImplementedError("write your pallas kernel here")


# speedup vs baseline: 2.1701x; 2.1701x over previous
import functools

import jax
import jax.numpy as jnp
from jax import lax
from jax.experimental import pallas as pl
from jax.experimental.pallas import tpu as pltpu
from jax.experimental.pallas import tpu_sc as plsc

_B, _T, _NCODE, _DC = 1024, 50, 26, 16
_V, _D = 1000000, 64
_BT = _B * _T
_DOUT = _D + _DC

_NCORES = 2
_NSUB = 16
_NWORK = _NCORES * _NSUB
_TOKW = _BT // _NWORK
_CHUNK = 32
_NCHUNK = _TOKW // _CHUNK
_IDXC = _CHUNK * _NCODE
_GS = 104
_NG = _IDXC // _GS
_LANE = 16
_NV = _D // _LANE


def _build():
    mesh = plsc.VectorSubcoreMesh(core_axis_name="c", subcore_axis_name="s")

    @functools.partial(
        pl.kernel,
        out_type=jax.ShapeDtypeStruct((_BT, _DOUT), jnp.float32),
        mesh=mesh,
        compiler_params=pltpu.CompilerParams(use_tc_tiling_on_sc=False),
        scratch_types=[
            pltpu.VMEM((2, _NG, _GS), jnp.int32),
            pltpu.VMEM((2, _IDXC, _D), jnp.float32),
            pltpu.VMEM((_CHUNK, _DC), jnp.float32),
            pltpu.VMEM((_CHUNK, _DOUT), jnp.float32),
            pltpu.SemaphoreType.DMA,
            pltpu.SemaphoreType.DMA,
        ],
    )
    def ker(cat_hbm, cont_hbm, table_hbm, out_hbm,
            idx_v, rows_v, cont_v, out_v, sem0, sem1):
        wid = lax.axis_index("s") * _NCORES + lax.axis_index("c")
        tok0 = wid * _TOKW
        row0 = wid * (_TOKW * _NCODE // _GS)
        sems = (sem0, sem1)

        def fire(g, slot):
            pltpu.sync_copy(cat_hbm.at[pl.ds(row0 + g * _NG, _NG)],
                            idx_v.at[slot])
            for j in range(_NG):
                pltpu.async_copy(
                    table_hbm.at[idx_v.at[slot, j]],
                    rows_v.at[slot, pl.ds(j * _GS, _GS)],
                    sems[slot])

        def wait(slot):
            for j in range(_NG):
                pltpu.make_async_copy(
                    table_hbm.at[idx_v.at[slot, j]],
                    rows_v.at[slot, pl.ds(j * _GS, _GS)],
                    sems[slot]).wait()

        def compute_store(g, slot):
            base = tok0 + g * _CHUNK
            pltpu.sync_copy(cont_hbm.at[pl.ds(base, _CHUNK)], cont_v)

            def body(i, carry):
                rb = i * _NCODE
                accs = [rows_v[slot, rb, pl.ds(c * _LANE, _LANE)]
                        for c in range(_NV)]
                for r in range(1, _NCODE):
                    for c in range(_NV):
                        accs[c] = accs[c] + rows_v[slot, rb + r,
                                                   pl.ds(c * _LANE, _LANE)]
                for c in range(_NV):
                    out_v[i, pl.ds(c * _LANE, _LANE)] = accs[c]
                out_v[i, pl.ds(_D, _DC)] = cont_v[i, :]
                return carry

            lax.fori_loop(0, _CHUNK, body, 0)
            pltpu.sync_copy(out_v, out_hbm.at[pl.ds(base, _CHUNK)])

        fire(0, 0)

        def outer(g2, carry):
            g = g2 * 2
            fire(g + 1, 1)
            wait(0)
            compute_store(g, 0)
            fire(g + 2, 0)
            wait(1)
            compute_store(g + 1, 1)
            return carry

        lax.fori_loop(0, _NCHUNK // 2 - 1, outer, 0)

        g_last = _NCHUNK - 2
        fire(g_last + 1, 1)
        wait(0)
        compute_store(g_last, 0)
        wait(1)
        compute_store(g_last + 1, 1)

    return ker


_EMBED_SUM = _build()


def kernel(ContTensor, CatTensor, LabelTensor, DoseTensor, TimeDiffTensor,
           VTensor, VancoElTensor, PtList, LengList, embed_weight):
    cat2d = CatTensor.reshape(_BT * _NCODE // _GS, _GS).astype(jnp.int32)
    cont2d = ContTensor.reshape(_BT, _DC)
    out = _EMBED_SUM(cat2d, cont2d, embed_weight)
    outEmb = out.reshape(_B, _T, _DOUT)
    return (outEmb, LabelTensor, LengList, DoseTensor, TimeDiffTensor,
            VTensor, VancoElTensor, PtList)


# speedup vs baseline: 2.8265x; 1.3025x over previous
import functools

import jax
import jax.numpy as jnp
from jax import lax
from jax.experimental import pallas as pl
from jax.experimental.pallas import tpu as pltpu
from jax.experimental.pallas import tpu_sc as plsc

_B, _T, _NCODE, _DC = 1024, 50, 26, 16
_V, _D = 1000000, 64
_BT = _B * _T
_DOUT = _D + _DC

_CB = 2048
_TGRID = (_V + 2 * _CB - 1) // (2 * _CB)
_VPAD = _TGRID * 2 * _CB
_TAIL0 = (_TGRID - 1) * 2 * _CB

_NCORES = 2
_NSUB = 16
_NWORK = _NCORES * _NSUB
_TOKW = _BT // _NWORK
_CHUNK = 32
_NCHUNK = _TOKW // _CHUNK
_IDXC = _CHUNK * _NCODE
_GS = 104
_NG = _IDXC // _GS
_LANE = 16
_NV = _D // _LANE


def _relayout_body(a_ref, b_ref, y_ref):
    y_ref[...] = jnp.concatenate([a_ref[...].T, b_ref[...].T], axis=1)


_RELAYOUT = pl.pallas_call(
    _relayout_body,
    grid=(_TGRID,),
    in_specs=[
        pl.BlockSpec((_D, _CB), lambda k: (0, 2 * k)),
        pl.BlockSpec((_D, _CB),
                     lambda k: (0, jnp.minimum(2 * k + 1, 2 * _TGRID - 2))),
    ],
    out_specs=pl.BlockSpec((_CB, 128), lambda k: (k, 0)),
    out_shape=jax.ShapeDtypeStruct((_VPAD // 2, 128), jnp.float32),
)


def _permute_idx(t):
    return jnp.where(t < _TAIL0,
                     (t & ~4095) | ((t & 2047) << 1) | ((t >> 11) & 1),
                     2 * t - _TAIL0)


def _build():
    mesh = plsc.VectorSubcoreMesh(core_axis_name="c", subcore_axis_name="s")

    @functools.partial(
        pl.kernel,
        out_type=jax.ShapeDtypeStruct((_BT, _DOUT), jnp.float32),
        mesh=mesh,
        compiler_params=pltpu.CompilerParams(use_tc_tiling_on_sc=False),
        scratch_types=[
            pltpu.VMEM((2, _NG, _GS), jnp.int32),
            pltpu.VMEM((2, _IDXC, _D), jnp.float32),
            pltpu.VMEM((_CHUNK, _DC), jnp.float32),
            pltpu.VMEM((_CHUNK, _DOUT), jnp.float32),
            pltpu.SemaphoreType.DMA,
            pltpu.SemaphoreType.DMA,
        ],
    )
    def ker(cat_hbm, cont_hbm, table_hbm, out_hbm,
            idx_v, rows_v, cont_v, out_v, sem0, sem1):
        wid = lax.axis_index("s") * _NCORES + lax.axis_index("c")
        tok0 = wid * _TOKW
        row0 = wid * (_TOKW * _NCODE // _GS)
        sems = (sem0, sem1)

        def fire(g, slot):
            pltpu.sync_copy(cat_hbm.at[pl.ds(row0 + g * _NG, _NG)],
                            idx_v.at[slot])
            for j in range(_NG):
                pltpu.async_copy(
                    table_hbm.at[idx_v.at[slot, j]],
                    rows_v.at[slot, pl.ds(j * _GS, _GS)],
                    sems[slot])

        def wait(slot):
            for j in range(_NG):
                pltpu.make_async_copy(
                    table_hbm.at[idx_v.at[slot, j]],
                    rows_v.at[slot, pl.ds(j * _GS, _GS)],
                    sems[slot]).wait()

        def compute_store(g, slot):
            base = tok0 + g * _CHUNK
            pltpu.sync_copy(cont_hbm.at[pl.ds(base, _CHUNK)], cont_v)

            def body(i, carry):
                rb = i * _NCODE
                accs = [rows_v[slot, rb, pl.ds(c * _LANE, _LANE)]
                        for c in range(_NV)]
                for r in range(1, _NCODE):
                    for c in range(_NV):
                        accs[c] = accs[c] + rows_v[slot, rb + r,
                                                   pl.ds(c * _LANE, _LANE)]
                for c in range(_NV):
                    out_v[i, pl.ds(c * _LANE, _LANE)] = accs[c]
                out_v[i, pl.ds(_D, _DC)] = cont_v[i, :]
                return carry

            lax.fori_loop(0, _CHUNK, body, 0)
            pltpu.sync_copy(out_v, out_hbm.at[pl.ds(base, _CHUNK)])

        fire(0, 0)

        def outer(g2, carry):
            g = g2 * 2
            fire(g + 1, 1)
            wait(0)
            compute_store(g, 0)
            fire(g + 2, 0)
            wait(1)
            compute_store(g + 1, 1)
            return carry

        lax.fori_loop(0, _NCHUNK // 2 - 1, outer, 0)

        g_last = _NCHUNK - 2
        fire(g_last + 1, 1)
        wait(0)
        compute_store(g_last, 0)
        wait(1)
        compute_store(g_last + 1, 1)

    return ker


_EMBED_SUM = _build()


def kernel(ContTensor, CatTensor, LabelTensor, DoseTensor, TimeDiffTensor,
           VTensor, VancoElTensor, PtList, LengList, embed_weight):
    cat2d = _permute_idx(CatTensor.astype(jnp.int32)).reshape(
        _BT * _NCODE // _GS, _GS)
    cont2d = ContTensor.reshape(_BT, _DC)
    t_t = embed_weight.T
    t_lin = _RELAYOUT(t_t, t_t)
    out = _EMBED_SUM(cat2d, cont2d, t_lin.reshape(_VPAD, _D))
    outEmb = out.reshape(_B, _T, _DOUT)
    return (outEmb, LabelTensor, LengList, DoseTensor, TimeDiffTensor,
            VTensor, VancoElTensor, PtList)
